# Initial kernel scaffold; baseline (speedup 1.0000x reference)
import jax
import jax.numpy as jnp
from jax.experimental import pallas as pl


def kernel(x, edge_index_u, edge_index_u2, W_in, b_in, W_hid, b_hid, W_out, b_out):
    raise NotImplementedError("write your pallas kernel here")


# speedup vs baseline: 30.7516x; 30.7516x over previous
import functools

import jax
import jax.numpy as jnp
from jax import lax
from jax.experimental import pallas as pl
from jax.experimental.pallas import tpu as pltpu
from jax.experimental.pallas import tpu_sc as plsc

N = 10000
E = 320000
D_IN = 128
D_HID = 32
D_OUT = 64

NC = 2
NS = 16
B = 100
EW = E // NS
NBLK = EW // B
NP = 10240
RPS = NP // NS
DEGW = 16

@functools.cache
def _mesh():
    return plsc.VectorSubcoreMesh(
        core_axis_name="c", subcore_axis_name="s",
        num_cores=NC, num_subcores=NS)


def _fill(buf, rows, d, value):
    v = jnp.full((16,), value, jnp.float32)

    def body(r, carry):
        for c in range(d // 16):
            buf[r, pl.ds(c * 16, 16)] = v
        return carry

    lax.fori_loop(0, rows, body, 0)


def _make_deg_kernel():
    @functools.partial(
        pl.kernel,
        out_type=[jax.ShapeDtypeStruct((NP, DEGW), jnp.float32),
                  jax.ShapeDtypeStruct((NP, DEGW), jnp.float32)],
        mesh=_mesh(),
        compiler_params=pltpu.CompilerParams(use_tc_tiling_on_sc=False),
        scratch_types=[
            pltpu.VMEM((NBLK, B), jnp.int32),
            pltpu.VMEM((B, DEGW), jnp.float32),
            pltpu.VMEM((RPS, DEGW), jnp.float32),
            pltpu.VMEM_SHARED((NP, DEGW), jnp.float32),
            pltpu.SemaphoreType.DMA,
        ],
    )
    def deg_kernel(dstA, dstB, outA, outB, dst_v, ones_v, zbuf, acc, sem):
        cid = lax.axis_index("c")
        sid = lax.axis_index("s")
        stripe = pl.ds(sid * RPS, RPS)

        _fill(zbuf, RPS, DEGW, 0.0)
        _fill(ones_v, B, DEGW, 1.0)
        pltpu.sync_copy(zbuf, acc.at[stripe])
        plsc.subcore_barrier()

        def run(dst_ref):
            pltpu.sync_copy(dst_ref.at[sid], dst_v)

            def body(j, carry):
                pltpu.async_copy(ones_v, acc.at[dst_v.at[j]], sem,
                                 add=True).wait()
                return carry

            lax.fori_loop(0, NBLK, body, 0)

        @pl.when(cid == 0)
        def _():
            run(dstA)

        @pl.when(cid == 1)
        def _():
            run(dstB)

        plsc.subcore_barrier()

        @pl.when(cid == 0)
        def _():
            pltpu.sync_copy(acc.at[stripe], outA.at[stripe])

        @pl.when(cid == 1)
        def _():
            pltpu.sync_copy(acc.at[stripe], outB.at[stripe])

    return deg_kernel


def _make_scatter_kernel(d, stage_g=True):
    @functools.partial(
        pl.kernel,
        out_type=[jax.ShapeDtypeStruct((NP, d), jnp.float32),
                  jax.ShapeDtypeStruct((NP, d), jnp.float32)],
        mesh=_mesh(),
        compiler_params=pltpu.CompilerParams(use_tc_tiling_on_sc=False),
        scratch_types=[
            pltpu.VMEM((NBLK, B), jnp.int32),
            pltpu.VMEM((NBLK, B), jnp.int32),
            pltpu.VMEM((B, d), jnp.float32),
            pltpu.VMEM((RPS, d), jnp.float32),
            pltpu.VMEM_SHARED((NP, d) if stage_g else (8, d),
                              jnp.float32),
            pltpu.VMEM_SHARED((NP, d), jnp.float32),
            pltpu.SemaphoreType.DMA,
            pltpu.SemaphoreType.DMA,
        ],
    )
    def scatter_kernel(gA, srcA, dstA, gB, srcB, dstB, outA, outB,
                       src_v, dst_v, rows_v, zbuf, gtab, acc, sem_g, sem_s):
        cid = lax.axis_index("c")
        sid = lax.axis_index("s")
        stripe = pl.ds(sid * RPS, RPS)

        _fill(zbuf, RPS, d, 0.0)
        pltpu.sync_copy(zbuf, acc.at[stripe])

        if stage_g:
            @pl.when(cid == 0)
            def _():
                pltpu.sync_copy(gA.at[stripe], gtab.at[stripe])

            @pl.when(cid == 1)
            def _():
                pltpu.sync_copy(gB.at[stripe], gtab.at[stripe])

        plsc.subcore_barrier()

        def run(g_ref, src_ref, dst_ref):
            tab = gtab if stage_g else g_ref
            pltpu.sync_copy(src_ref.at[sid], src_v)
            pltpu.sync_copy(dst_ref.at[sid], dst_v)

            def body(j, carry):
                pltpu.async_copy(tab.at[src_v.at[j]], rows_v, sem_g).wait()
                pltpu.async_copy(rows_v, acc.at[dst_v.at[j]], sem_s,
                                 add=True).wait()
                return carry

            lax.fori_loop(0, NBLK, body, 0)

        @pl.when(cid == 0)
        def _():
            run(gA, srcA, dstA)

        @pl.when(cid == 1)
        def _():
            run(gB, srcB, dstB)

        plsc.subcore_barrier()

        @pl.when(cid == 0)
        def _():
            pltpu.sync_copy(acc.at[stripe], outA.at[stripe])

        @pl.when(cid == 1)
        def _():
            pltpu.sync_copy(acc.at[stripe], outB.at[stripe])

    return scatter_kernel


BN = 2000
GRID = (N // BN,)


def _dinv(deg_blk):
    return lax.rsqrt(deg_blk[:, 0:1] + 1.0)


def _tc_a_body(x_ref, w_ref, dgu_ref, dgv_ref, g1_ref, g2_ref):
    h = jnp.dot(x_ref[...], w_ref[...], preferred_element_type=jnp.float32)
    g1_ref[...] = _dinv(dgu_ref[...]) * h
    g2_ref[...] = _dinv(dgv_ref[...]) * h


def _tc_fuse_body(pA_ref, pB_ref, g1_ref, g2_ref, dgu_ref, dgv_ref,
                  w_ref, b_ref, o1_ref, o2_ref, *, second_dinv_u):
    du = _dinv(dgu_ref[...])
    dv = _dinv(dgv_ref[...])
    x1 = du * (pA_ref[...] + g1_ref[...]) + b_ref[...]
    x2 = dv * (pB_ref[...] + g2_ref[...]) + b_ref[...]
    n1 = jnp.maximum(jnp.sqrt(jnp.sum(x1 * x1, axis=1, keepdims=True)), 1e-12)
    n2 = jnp.maximum(jnp.sqrt(jnp.sum(x2 * x2, axis=1, keepdims=True)), 1e-12)
    sim = jnp.sum((x1 / n1) * (x2 / n2), axis=1, keepdims=True)
    mian = x1 + x2 * sim
    sup = x2 + x1 * sim
    h1 = jnp.dot(mian, w_ref[...], preferred_element_type=jnp.float32)
    h2 = jnp.dot(sup, w_ref[...], preferred_element_type=jnp.float32)
    o1_ref[...] = du * h1
    o2_ref[...] = (du if second_dinv_u else dv) * h2


def _tc_d_body(pA_ref, pB_ref, g1_ref, g2_ref, dgu_ref, b_ref, o_ref):
    du = _dinv(dgu_ref[...])
    x1 = du * (pA_ref[...] + g1_ref[...]) + b_ref[...]
    x2 = du * (pB_ref[...] + g2_ref[...]) + b_ref[...]
    o_ref[...] = jnp.concatenate([x1, x2], axis=1)


def _row_spec(d):
    return pl.BlockSpec((BN, d), lambda b: (b, 0))


def _full_spec(shape):
    return pl.BlockSpec(shape, lambda b: tuple(0 for _ in shape))


def kernel(x, edge_index_u, edge_index_u2, W_in, b_in, W_hid, b_hid,
           W_out, b_out):
    src_u = edge_index_u[0].reshape(NS, NBLK, B)
    dst_u = edge_index_u[1].reshape(NS, NBLK, B)
    src_v = edge_index_u2[0].reshape(NS, NBLK, B)
    dst_v = edge_index_u2[1].reshape(NS, NBLK, B)
    b_in2 = b_in.reshape(1, D_HID)
    b_hid2 = b_hid.reshape(1, D_HID)
    b_out2 = b_out.reshape(1, D_OUT)

    deg_u, deg_v = _make_deg_kernel()(dst_u, dst_v)

    g1, g2 = pl.pallas_call(
        _tc_a_body,
        grid=GRID,
        in_specs=[_row_spec(D_IN), _full_spec((D_IN, D_HID)),
                  _row_spec(DEGW), _row_spec(DEGW)],
        out_specs=[_row_spec(D_HID), _row_spec(D_HID)],
        out_shape=[jax.ShapeDtypeStruct((NP, D_HID), jnp.float32)] * 2,
    )(x, W_in, deg_u, deg_v)

    p1, p2 = _make_scatter_kernel(D_HID)(g1, src_u, dst_u, g2, src_v, dst_v)

    g1, g2 = pl.pallas_call(
        functools.partial(_tc_fuse_body, second_dinv_u=False),
        grid=GRID,
        in_specs=[_row_spec(D_HID)] * 4 + [_row_spec(DEGW)] * 2 +
                 [_full_spec((D_HID, D_HID)), _full_spec((1, D_HID))],
        out_specs=[_row_spec(D_HID)] * 2,
        out_shape=[jax.ShapeDtypeStruct((NP, D_HID), jnp.float32)] * 2,
    )(p1, p2, g1, g2, deg_u, deg_v, W_hid, b_in2)

    p1, p2 = _make_scatter_kernel(D_HID)(g1, src_u, dst_u, g2, src_v, dst_v)

    g1, g2 = pl.pallas_call(
        functools.partial(_tc_fuse_body, second_dinv_u=True),
        grid=GRID,
        in_specs=[_row_spec(D_HID)] * 4 + [_row_spec(DEGW)] * 2 +
                 [_full_spec((D_HID, D_OUT)), _full_spec((1, D_HID))],
        out_specs=[_row_spec(D_OUT)] * 2,
        out_shape=[jax.ShapeDtypeStruct((NP, D_OUT), jnp.float32)] * 2,
    )(p1, p2, g1, g2, deg_u, deg_v, W_out, b_hid2)

    p1, p2 = _make_scatter_kernel(D_OUT, stage_g=False)(g1, src_u, dst_u, g2, src_u, dst_u)

    out = pl.pallas_call(
        _tc_d_body,
        grid=GRID,
        in_specs=[_row_spec(D_OUT)] * 4 + [_row_spec(DEGW),
                  _full_spec((1, D_OUT))],
        out_specs=_row_spec(2 * D_OUT),
        out_shape=jax.ShapeDtypeStruct((N, 2 * D_OUT), jnp.float32),
    )(p1, p2, g1, g2, deg_u, b_out2)

    return out


# speedup vs baseline: 48.4135x; 1.5743x over previous
import functools

import jax
import jax.numpy as jnp
from jax import lax
from jax.experimental import pallas as pl
from jax.experimental.pallas import tpu as pltpu
from jax.experimental.pallas import tpu_sc as plsc

N = 10000
E = 320000
D_IN = 128
D_HID = 32
D_OUT = 64

NC = 2
NS = 16
B = 125
EW = E // NS
NBLK = EW // B
NP = 10240
RPS = NP // NS
DEGW = 16
NBUF = 4
ZR = 128

@functools.cache
def _mesh():
    return plsc.VectorSubcoreMesh(
        core_axis_name="c", subcore_axis_name="s",
        num_cores=NC, num_subcores=NS)


def _fill(buf, rows, d, value):
    v = jnp.full((16,), value, jnp.float32)

    def body(r, carry):
        for c in range(d // 16):
            buf[r, pl.ds(c * 16, 16)] = v
        return carry

    lax.fori_loop(0, rows, body, 0)


def _make_deg_kernel():
    @functools.partial(
        pl.kernel,
        out_type=[jax.ShapeDtypeStruct((NP, DEGW), jnp.float32),
                  jax.ShapeDtypeStruct((NP, DEGW), jnp.float32)],
        mesh=_mesh(),
        compiler_params=pltpu.CompilerParams(use_tc_tiling_on_sc=False),
        scratch_types=[
            pltpu.VMEM((NBLK, B), jnp.int32),
            pltpu.VMEM((B, DEGW), jnp.float32),
            pltpu.VMEM((ZR, DEGW), jnp.float32),
            pltpu.VMEM_SHARED((NP, DEGW), jnp.float32),
            pltpu.SemaphoreType.DMA,
        ],
    )
    def deg_kernel(dstA, dstB, outA, outB, dst_v, ones_v, zbuf, acc, sem):
        cid = lax.axis_index("c")
        sid = lax.axis_index("s")
        stripe = pl.ds(sid * RPS, RPS)

        _fill(zbuf, ZR, DEGW, 0.0)
        _fill(ones_v, B, DEGW, 1.0)
        for z in range(RPS // ZR):
            pltpu.sync_copy(zbuf, acc.at[pl.ds(sid * RPS + z * ZR, ZR)])
        plsc.subcore_barrier()

        def run(dst_ref):
            pltpu.sync_copy(dst_ref.at[sid], dst_v)

            def body(j, carry):
                pltpu.async_copy(ones_v, acc.at[dst_v.at[j]], sem, add=True)
                return carry

            lax.fori_loop(0, NBLK, body, 0)

            def drain(j, carry):
                pltpu.make_async_copy(ones_v, acc.at[dst_v.at[j]], sem).wait()
                return carry

            lax.fori_loop(0, NBLK, drain, 0)

        @pl.when(cid == 0)
        def _():
            run(dstA)

        @pl.when(cid == 1)
        def _():
            run(dstB)

        plsc.subcore_barrier()

        @pl.when(cid == 0)
        def _():
            pltpu.sync_copy(acc.at[stripe], outA.at[stripe])

        @pl.when(cid == 1)
        def _():
            pltpu.sync_copy(acc.at[stripe], outB.at[stripe])

    return deg_kernel


def _make_scatter_kernel(d, stage_g=True):
    @functools.partial(
        pl.kernel,
        out_type=[jax.ShapeDtypeStruct((NP, d), jnp.float32),
                  jax.ShapeDtypeStruct((NP, d), jnp.float32)],
        mesh=_mesh(),
        compiler_params=pltpu.CompilerParams(use_tc_tiling_on_sc=False),
        scratch_types=[
            pltpu.VMEM((NBLK, B), jnp.int32),
            pltpu.VMEM((NBLK, B), jnp.int32),
            [pltpu.VMEM((B, d), jnp.float32)] * NBUF,
            pltpu.VMEM((ZR, d), jnp.float32),
            pltpu.VMEM_SHARED((NP, d) if stage_g else (8, d),
                              jnp.float32),
            pltpu.VMEM_SHARED((NP, d), jnp.float32),
            [pltpu.SemaphoreType.DMA] * NBUF,
            [pltpu.SemaphoreType.DMA] * NBUF,
        ],
    )
    def scatter_kernel(gA, srcA, dstA, gB, srcB, dstB, outA, outB,
                       src_v, dst_v, rows, zbuf, gtab, acc, sem_g, sem_s):
        cid = lax.axis_index("c")
        sid = lax.axis_index("s")
        stripe = pl.ds(sid * RPS, RPS)

        _fill(zbuf, ZR, d, 0.0)
        for z in range(RPS // ZR):
            pltpu.sync_copy(zbuf, acc.at[pl.ds(sid * RPS + z * ZR, ZR)])

        if stage_g:
            @pl.when(cid == 0)
            def _():
                pltpu.sync_copy(gA.at[stripe], gtab.at[stripe])

            @pl.when(cid == 1)
            def _():
                pltpu.sync_copy(gB.at[stripe], gtab.at[stripe])

        plsc.subcore_barrier()

        def run(g_ref, src_ref, dst_ref):
            tab = gtab if stage_g else g_ref
            pltpu.sync_copy(src_ref.at[sid], src_v)
            pltpu.sync_copy(dst_ref.at[sid], dst_v)

            def body(i, carry):
                for k in range(NBUF):
                    j = NBUF * i + k

                    @pl.when(i > 0)
                    def _(k=k, j=j):
                        pltpu.make_async_copy(
                            rows[k], acc.at[dst_v.at[j - NBUF]],
                            sem_s[k]).wait()

                    pltpu.async_copy(tab.at[src_v.at[j]], rows[k], sem_g[k])
                for k in range(NBUF):
                    j = NBUF * i + k
                    pltpu.make_async_copy(tab.at[src_v.at[j]], rows[k],
                                          sem_g[k]).wait()
                    pltpu.async_copy(rows[k], acc.at[dst_v.at[j]], sem_s[k],
                                     add=True)
                return carry

            lax.fori_loop(0, NBLK // NBUF, body, 0)
            for k in range(NBUF):
                pltpu.make_async_copy(rows[k], acc.at[dst_v.at[NBLK - NBUF + k]],
                                      sem_s[k]).wait()

        @pl.when(cid == 0)
        def _():
            run(gA, srcA, dstA)

        @pl.when(cid == 1)
        def _():
            run(gB, srcB, dstB)

        plsc.subcore_barrier()

        @pl.when(cid == 0)
        def _():
            pltpu.sync_copy(acc.at[stripe], outA.at[stripe])

        @pl.when(cid == 1)
        def _():
            pltpu.sync_copy(acc.at[stripe], outB.at[stripe])

    return scatter_kernel


BN = 2000
GRID = (N // BN,)


def _dinv(deg_blk):
    return lax.rsqrt(deg_blk[:, 0:1] + 1.0)


def _tc_mm_body(x_ref, w_ref, h_ref):
    h_ref[...] = jnp.dot(x_ref[...], w_ref[...],
                         preferred_element_type=jnp.float32)


def _tc_scale_body(h_ref, dgu_ref, dgv_ref, g1_ref, g2_ref):
    h = h_ref[...]
    g1_ref[...] = _dinv(dgu_ref[...]) * h
    g2_ref[...] = _dinv(dgv_ref[...]) * h


def _tc_fuse_body(pA_ref, pB_ref, g1_ref, g2_ref, dgu_ref, dgv_ref,
                  w_ref, b_ref, o1_ref, o2_ref, *, second_dinv_u):
    du = _dinv(dgu_ref[...])
    dv = _dinv(dgv_ref[...])
    x1 = du * (pA_ref[...] + g1_ref[...]) + b_ref[...]
    x2 = dv * (pB_ref[...] + g2_ref[...]) + b_ref[...]
    ss1 = jnp.maximum(jnp.sum(x1 * x1, axis=1, keepdims=True), 1e-24)
    ss2 = jnp.maximum(jnp.sum(x2 * x2, axis=1, keepdims=True), 1e-24)
    dot = jnp.sum(x1 * x2, axis=1, keepdims=True)
    sim = dot * lax.rsqrt(ss1 * ss2)
    mian = x1 + x2 * sim
    sup = x2 + x1 * sim
    h1 = jnp.dot(mian, w_ref[...], preferred_element_type=jnp.float32)
    h2 = jnp.dot(sup, w_ref[...], preferred_element_type=jnp.float32)
    o1_ref[...] = du * h1
    o2_ref[...] = (du if second_dinv_u else dv) * h2


def _tc_d_body(pA_ref, pB_ref, g1_ref, g2_ref, dgu_ref, b_ref, o_ref):
    du = _dinv(dgu_ref[...])
    x1 = du * (pA_ref[...] + g1_ref[...]) + b_ref[...]
    x2 = du * (pB_ref[...] + g2_ref[...]) + b_ref[...]
    o_ref[...] = jnp.concatenate([x1, x2], axis=1)


def _row_spec(d):
    return pl.BlockSpec((BN, d), lambda b: (b, 0))


def _full_spec(shape):
    return pl.BlockSpec(shape, lambda b: tuple(0 for _ in shape))


def kernel(x, edge_index_u, edge_index_u2, W_in, b_in, W_hid, b_hid,
           W_out, b_out):
    src_u = edge_index_u[0].reshape(NS, NBLK, B)
    dst_u = edge_index_u[1].reshape(NS, NBLK, B)
    src_v = edge_index_u2[0].reshape(NS, NBLK, B)
    dst_v = edge_index_u2[1].reshape(NS, NBLK, B)
    b_in2 = b_in.reshape(1, D_HID)
    b_hid2 = b_hid.reshape(1, D_HID)
    b_out2 = b_out.reshape(1, D_OUT)

    deg_u, deg_v = _make_deg_kernel()(dst_u, dst_v)
    scat32 = _make_scatter_kernel(D_HID, stage_g=False)

    h = pl.pallas_call(
        _tc_mm_body,
        grid=GRID,
        in_specs=[_row_spec(D_IN), _full_spec((D_IN, D_HID))],
        out_specs=_row_spec(D_HID),
        out_shape=jax.ShapeDtypeStruct((NP, D_HID), jnp.float32),
    )(x, W_in)

    g1, g2 = pl.pallas_call(
        _tc_scale_body,
        grid=GRID,
        in_specs=[_row_spec(D_HID), _row_spec(DEGW), _row_spec(DEGW)],
        out_specs=[_row_spec(D_HID), _row_spec(D_HID)],
        out_shape=[jax.ShapeDtypeStruct((NP, D_HID), jnp.float32)] * 2,
    )(h, deg_u, deg_v)

    p1, p2 = scat32(g1, src_u, dst_u, g2, src_v, dst_v)

    g1, g2 = pl.pallas_call(
        functools.partial(_tc_fuse_body, second_dinv_u=False),
        grid=GRID,
        in_specs=[_row_spec(D_HID)] * 4 + [_row_spec(DEGW)] * 2 +
                 [_full_spec((D_HID, D_HID)), _full_spec((1, D_HID))],
        out_specs=[_row_spec(D_HID)] * 2,
        out_shape=[jax.ShapeDtypeStruct((NP, D_HID), jnp.float32)] * 2,
    )(p1, p2, g1, g2, deg_u, deg_v, W_hid, b_in2)

    p1, p2 = scat32(g1, src_u, dst_u, g2, src_v, dst_v)

    g1, g2 = pl.pallas_call(
        functools.partial(_tc_fuse_body, second_dinv_u=True),
        grid=GRID,
        in_specs=[_row_spec(D_HID)] * 4 + [_row_spec(DEGW)] * 2 +
                 [_full_spec((D_HID, D_OUT)), _full_spec((1, D_HID))],
        out_specs=[_row_spec(D_OUT)] * 2,
        out_shape=[jax.ShapeDtypeStruct((NP, D_OUT), jnp.float32)] * 2,
    )(p1, p2, g1, g2, deg_u, deg_v, W_out, b_hid2)

    p1, p2 = _make_scatter_kernel(D_OUT, stage_g=False)(g1, src_u, dst_u, g2, src_u, dst_u)

    out = pl.pallas_call(
        _tc_d_body,
        grid=GRID,
        in_specs=[_row_spec(D_OUT)] * 4 + [_row_spec(DEGW),
                  _full_spec((1, D_OUT))],
        out_specs=_row_spec(2 * D_OUT),
        out_shape=jax.ShapeDtypeStruct((N, 2 * D_OUT), jnp.float32),
    )(p1, p2, g1, g2, deg_u, b_out2)

    return out


# speedup vs baseline: 50.1696x; 1.0363x over previous
import functools

import jax
import jax.numpy as jnp
from jax import lax
from jax.experimental import pallas as pl
from jax.experimental.pallas import tpu as pltpu
from jax.experimental.pallas import tpu_sc as plsc

N = 10000
E = 320000
D_IN = 128
D_HID = 32
D_OUT = 64

NC = 2
NS = 16
B = 125
EW = E // NS
NBLK = EW // B
NP = 10240
RPS = NP // NS
DEGW = 16
NBUF = 4
ZR = 128

@functools.cache
def _mesh():
    return plsc.VectorSubcoreMesh(
        core_axis_name="c", subcore_axis_name="s",
        num_cores=NC, num_subcores=NS)


def _fill(buf, rows, d, value):
    v = jnp.full((16,), value, jnp.float32)

    def body(r, carry):
        for c in range(d // 16):
            buf[r, pl.ds(c * 16, 16)] = v
        return carry

    lax.fori_loop(0, rows, body, 0)


def _make_deg_kernel():
    @functools.partial(
        pl.kernel,
        out_type=[jax.ShapeDtypeStruct((NP, DEGW), jnp.float32),
                  jax.ShapeDtypeStruct((NP, DEGW), jnp.float32)],
        mesh=_mesh(),
        compiler_params=pltpu.CompilerParams(use_tc_tiling_on_sc=False),
        scratch_types=[
            pltpu.VMEM((NBLK, B), jnp.int32),
            pltpu.VMEM((B, DEGW), jnp.float32),
            pltpu.VMEM((ZR, DEGW), jnp.float32),
            pltpu.VMEM_SHARED((NP, DEGW), jnp.float32),
            pltpu.SemaphoreType.DMA,
        ],
    )
    def deg_kernel(dstA, dstB, outA, outB, dst_v, ones_v, zbuf, acc, sem):
        cid = lax.axis_index("c")
        sid = lax.axis_index("s")
        stripe = pl.ds(sid * RPS, RPS)

        _fill(zbuf, ZR, DEGW, 0.0)
        _fill(ones_v, B, DEGW, 1.0)
        for z in range(RPS // ZR):
            pltpu.sync_copy(zbuf, acc.at[pl.ds(sid * RPS + z * ZR, ZR)])
        plsc.subcore_barrier()

        def run(dst_ref):
            pltpu.sync_copy(dst_ref.at[sid], dst_v)

            def body(j, carry):
                pltpu.async_copy(ones_v, acc.at[dst_v.at[j]], sem, add=True)
                return carry

            lax.fori_loop(0, NBLK, body, 0)

            def drain(j, carry):
                pltpu.make_async_copy(ones_v, acc.at[dst_v.at[j]], sem).wait()
                return carry

            lax.fori_loop(0, NBLK, drain, 0)

        @pl.when(cid == 0)
        def _():
            run(dstA)

        @pl.when(cid == 1)
        def _():
            run(dstB)

        plsc.subcore_barrier()

        @pl.when(cid == 0)
        def _():
            pltpu.sync_copy(acc.at[stripe], outA.at[stripe])

        @pl.when(cid == 1)
        def _():
            pltpu.sync_copy(acc.at[stripe], outB.at[stripe])

    return deg_kernel


def _make_scatter_kernel(d, stage_g=True, nbuf=NBUF):
    @functools.partial(
        pl.kernel,
        out_type=[jax.ShapeDtypeStruct((NP, d), jnp.float32),
                  jax.ShapeDtypeStruct((NP, d), jnp.float32)],
        mesh=_mesh(),
        compiler_params=pltpu.CompilerParams(use_tc_tiling_on_sc=False),
        scratch_types=[
            pltpu.VMEM((NBLK, B), jnp.int32),
            pltpu.VMEM((NBLK, B), jnp.int32),
            [pltpu.VMEM((B, d), jnp.float32)] * nbuf,
            pltpu.VMEM((ZR, d), jnp.float32),
            pltpu.VMEM_SHARED((NP, d) if stage_g else (8, d),
                              jnp.float32),
            pltpu.VMEM_SHARED((NP, d), jnp.float32),
            [pltpu.SemaphoreType.DMA] * nbuf,
            [pltpu.SemaphoreType.DMA] * nbuf,
        ],
    )
    def scatter_kernel(gA, srcA, dstA, gB, srcB, dstB, outA, outB,
                       src_v, dst_v, rows, zbuf, gtab, acc, sem_g, sem_s):
        cid = lax.axis_index("c")
        sid = lax.axis_index("s")
        stripe = pl.ds(sid * RPS, RPS)

        _fill(zbuf, ZR, d, 0.0)
        for z in range(RPS // ZR):
            pltpu.sync_copy(zbuf, acc.at[pl.ds(sid * RPS + z * ZR, ZR)])

        if stage_g:
            @pl.when(cid == 0)
            def _():
                pltpu.sync_copy(gA.at[stripe], gtab.at[stripe])

            @pl.when(cid == 1)
            def _():
                pltpu.sync_copy(gB.at[stripe], gtab.at[stripe])

        plsc.subcore_barrier()

        def run(g_ref, src_ref, dst_ref):
            tab = gtab if stage_g else g_ref
            pltpu.sync_copy(src_ref.at[sid], src_v)
            pltpu.sync_copy(dst_ref.at[sid], dst_v)

            def body(i, carry):
                for k in range(nbuf):
                    j = nbuf * i + k

                    @pl.when(i > 0)
                    def _(k=k, j=j):
                        pltpu.make_async_copy(
                            rows[k], acc.at[dst_v.at[j - nbuf]],
                            sem_s[k]).wait()

                    pltpu.async_copy(tab.at[src_v.at[j]], rows[k], sem_g[k])
                for k in range(nbuf):
                    j = nbuf * i + k
                    pltpu.make_async_copy(tab.at[src_v.at[j]], rows[k],
                                          sem_g[k]).wait()
                    pltpu.async_copy(rows[k], acc.at[dst_v.at[j]], sem_s[k],
                                     add=True)
                return carry

            lax.fori_loop(0, NBLK // nbuf, body, 0)
            for k in range(nbuf):
                pltpu.make_async_copy(rows[k], acc.at[dst_v.at[NBLK - nbuf + k]],
                                      sem_s[k]).wait()

        @pl.when(cid == 0)
        def _():
            run(gA, srcA, dstA)

        @pl.when(cid == 1)
        def _():
            run(gB, srcB, dstB)

        plsc.subcore_barrier()

        @pl.when(cid == 0)
        def _():
            pltpu.sync_copy(acc.at[stripe], outA.at[stripe])

        @pl.when(cid == 1)
        def _():
            pltpu.sync_copy(acc.at[stripe], outB.at[stripe])

    return scatter_kernel


BN = 2000
GRID = (N // BN,)


def _dinv(deg_blk):
    return lax.rsqrt(deg_blk[:, 0:1] + 1.0)


def _tc_mm_body(x_ref, w_ref, h_ref):
    h_ref[...] = jnp.dot(x_ref[...], w_ref[...],
                         preferred_element_type=jnp.float32)


def _tc_scale_body(h_ref, dgu_ref, dgv_ref, g1_ref, g2_ref):
    h = h_ref[...]
    g1_ref[...] = _dinv(dgu_ref[...]) * h
    g2_ref[...] = _dinv(dgv_ref[...]) * h


def _tc_fuse_body(pA_ref, pB_ref, g1_ref, g2_ref, dgu_ref, dgv_ref,
                  w_ref, b_ref, o1_ref, o2_ref, *, second_dinv_u):
    du = _dinv(dgu_ref[...])
    dv = _dinv(dgv_ref[...])
    x1 = du * (pA_ref[...] + g1_ref[...]) + b_ref[...]
    x2 = dv * (pB_ref[...] + g2_ref[...]) + b_ref[...]
    ss1 = jnp.maximum(jnp.sum(x1 * x1, axis=1, keepdims=True), 1e-24)
    ss2 = jnp.maximum(jnp.sum(x2 * x2, axis=1, keepdims=True), 1e-24)
    dot = jnp.sum(x1 * x2, axis=1, keepdims=True)
    sim = dot * lax.rsqrt(ss1 * ss2)
    mian = x1 + x2 * sim
    sup = x2 + x1 * sim
    h1 = jnp.dot(mian, w_ref[...], preferred_element_type=jnp.float32)
    h2 = jnp.dot(sup, w_ref[...], preferred_element_type=jnp.float32)
    o1_ref[...] = du * h1
    o2_ref[...] = (du if second_dinv_u else dv) * h2


def _tc_d_body(pA_ref, pB_ref, g1_ref, g2_ref, dgu_ref, b_ref, o_ref):
    du = _dinv(dgu_ref[...])
    x1 = du * (pA_ref[...] + g1_ref[...]) + b_ref[...]
    x2 = du * (pB_ref[...] + g2_ref[...]) + b_ref[...]
    o_ref[...] = jnp.concatenate([x1, x2], axis=1)


def _row_spec(d):
    return pl.BlockSpec((BN, d), lambda b: (b, 0))


def _full_spec(shape):
    return pl.BlockSpec(shape, lambda b: tuple(0 for _ in shape))


def kernel(x, edge_index_u, edge_index_u2, W_in, b_in, W_hid, b_hid,
           W_out, b_out):
    src_u = edge_index_u[0].reshape(NS, NBLK, B)
    dst_u = edge_index_u[1].reshape(NS, NBLK, B)
    src_v = edge_index_u2[0].reshape(NS, NBLK, B)
    dst_v = edge_index_u2[1].reshape(NS, NBLK, B)
    b_in2 = b_in.reshape(1, D_HID)
    b_hid2 = b_hid.reshape(1, D_HID)
    b_out2 = b_out.reshape(1, D_OUT)

    deg_u, deg_v = _make_deg_kernel()(dst_u, dst_v)
    scat32 = _make_scatter_kernel(D_HID, stage_g=False, nbuf=8)

    h = pl.pallas_call(
        _tc_mm_body,
        grid=GRID,
        in_specs=[_row_spec(D_IN), _full_spec((D_IN, D_HID))],
        out_specs=_row_spec(D_HID),
        out_shape=jax.ShapeDtypeStruct((NP, D_HID), jnp.float32),
    )(x, W_in)

    g1, g2 = pl.pallas_call(
        _tc_scale_body,
        grid=GRID,
        in_specs=[_row_spec(D_HID), _row_spec(DEGW), _row_spec(DEGW)],
        out_specs=[_row_spec(D_HID), _row_spec(D_HID)],
        out_shape=[jax.ShapeDtypeStruct((NP, D_HID), jnp.float32)] * 2,
    )(h, deg_u, deg_v)

    p1, p2 = scat32(g1, src_u, dst_u, g2, src_v, dst_v)

    g1, g2 = pl.pallas_call(
        functools.partial(_tc_fuse_body, second_dinv_u=False),
        grid=GRID,
        in_specs=[_row_spec(D_HID)] * 4 + [_row_spec(DEGW)] * 2 +
                 [_full_spec((D_HID, D_HID)), _full_spec((1, D_HID))],
        out_specs=[_row_spec(D_HID)] * 2,
        out_shape=[jax.ShapeDtypeStruct((NP, D_HID), jnp.float32)] * 2,
    )(p1, p2, g1, g2, deg_u, deg_v, W_hid, b_in2)

    p1, p2 = scat32(g1, src_u, dst_u, g2, src_v, dst_v)

    g1, g2 = pl.pallas_call(
        functools.partial(_tc_fuse_body, second_dinv_u=True),
        grid=GRID,
        in_specs=[_row_spec(D_HID)] * 4 + [_row_spec(DEGW)] * 2 +
                 [_full_spec((D_HID, D_OUT)), _full_spec((1, D_HID))],
        out_specs=[_row_spec(D_OUT)] * 2,
        out_shape=[jax.ShapeDtypeStruct((NP, D_OUT), jnp.float32)] * 2,
    )(p1, p2, g1, g2, deg_u, deg_v, W_out, b_hid2)

    p1, p2 = _make_scatter_kernel(D_OUT, stage_g=False)(g1, src_u, dst_u, g2, src_u, dst_u)

    out = pl.pallas_call(
        _tc_d_body,
        grid=GRID,
        in_specs=[_row_spec(D_OUT)] * 4 + [_row_spec(DEGW),
                  _full_spec((1, D_OUT))],
        out_specs=_row_spec(2 * D_OUT),
        out_shape=jax.ShapeDtypeStruct((N, 2 * D_OUT), jnp.float32),
    )(p1, p2, g1, g2, deg_u, b_out2)

    return out
